# baseline (device time: 34583 ns/iter reference)
import jax
import jax.numpy as jnp
from jax import lax
from jax.experimental import pallas as pl
from jax.experimental.pallas import tpu as pltpu


def kernel(x, dy):
    k, m = x.shape
    _, f = dy.shape
    m_half = m // 2
    f_half = f // 2

    def body(x_ref, dy_ref, out_ref, c_send, rs_recv, r_ref, ag_recv, sems):
        my_x = lax.axis_index("x")
        my_y = lax.axis_index("y")
        is_x0 = my_x == 0
        is_y0 = my_y == 0

        barrier = pltpu.get_barrier_semaphore()
        pl.semaphore_signal(barrier, inc=1, device_id=(1 - my_x, my_y),
                            device_id_type=pl.DeviceIdType.MESH)
        pl.semaphore_signal(barrier, inc=1, device_id=(my_x, 1 - my_y),
                            device_id_type=pl.DeviceIdType.MESH)
        pl.semaphore_wait(barrier, 2)

        dy_half = jnp.where(is_x0, dy_ref[:, :f_half], dy_ref[:, f_half:])
        x_send = jnp.where(is_y0, x_ref[:, m_half:], x_ref[:, :m_half])
        x_keep = jnp.where(is_y0, x_ref[:, :m_half], x_ref[:, m_half:])

        c_send[...] = lax.dot_general(
            x_send, dy_half, (((0,), (0,)), ((), ())),
            preferred_element_type=jnp.float32)

        rdma1 = pltpu.make_async_remote_copy(
            src_ref=c_send, dst_ref=rs_recv,
            send_sem=sems.at[0], recv_sem=sems.at[1],
            device_id=(my_x, 1 - my_y), device_id_type=pl.DeviceIdType.MESH)
        rdma1.start()

        c_keep = lax.dot_general(
            x_keep, dy_half, (((0,), (0,)), ((), ())),
            preferred_element_type=jnp.float32)

        rdma1.wait()
        r_ref[...] = c_keep + rs_recv[...]

        rdma2 = pltpu.make_async_remote_copy(
            src_ref=r_ref, dst_ref=ag_recv,
            send_sem=sems.at[2], recv_sem=sems.at[3],
            device_id=(1 - my_x, my_y), device_id_type=pl.DeviceIdType.MESH)
        rdma2.start()

        @pl.when(is_x0)
        def _():
            out_ref[:, :f_half] = r_ref[...]

        @pl.when(~is_x0)
        def _():
            out_ref[:, f_half:] = r_ref[...]

        rdma2.wait()

        @pl.when(is_x0)
        def _():
            out_ref[:, f_half:] = ag_recv[...]

        @pl.when(~is_x0)
        def _():
            out_ref[:, :f_half] = ag_recv[...]

    return pl.pallas_call(
        body,
        out_shape=jax.ShapeDtypeStruct((m_half, f), jnp.float32),
        in_specs=[pl.BlockSpec(memory_space=pltpu.VMEM),
                  pl.BlockSpec(memory_space=pltpu.VMEM)],
        out_specs=pl.BlockSpec(memory_space=pltpu.VMEM),
        scratch_shapes=[
            pltpu.VMEM((m_half, f_half), jnp.float32),
            pltpu.VMEM((m_half, f_half), jnp.float32),
            pltpu.VMEM((m_half, f_half), jnp.float32),
            pltpu.VMEM((m_half, f_half), jnp.float32),
            pltpu.SemaphoreType.DMA((4,)),
        ],
        compiler_params=pltpu.CompilerParams(collective_id=0),
    )(x, dy)


# device time: 24423 ns/iter; 1.4160x vs baseline; 1.4160x over previous
import jax
import jax.numpy as jnp
from jax import lax
from jax.experimental import pallas as pl
from jax.experimental.pallas import tpu as pltpu

NC = 8


def kernel(x, dy):
    k, m = x.shape
    _, f = dy.shape
    m_half = m // 2
    f_half = f // 2
    fc = f_half // NC

    def body(x_ref, dy_ref, out_ref, c_send, rs_recv, r_ref, ag_recv,
             sems1_s, sems1_r, sems2_s, sems2_r):
        my_x = lax.axis_index("x")
        my_y = lax.axis_index("y")
        is_x0 = my_x == 0
        is_y0 = my_y == 0

        barrier = pltpu.get_barrier_semaphore()
        pl.semaphore_signal(barrier, inc=1, device_id=(1 - my_x, my_y),
                            device_id_type=pl.DeviceIdType.MESH)
        pl.semaphore_signal(barrier, inc=1, device_id=(my_x, 1 - my_y),
                            device_id_type=pl.DeviceIdType.MESH)
        pl.semaphore_wait(barrier, 2)

        x_send = jnp.where(is_y0, x_ref[:, m_half:], x_ref[:, :m_half])
        x_keep = jnp.where(is_y0, x_ref[:, :m_half], x_ref[:, m_half:])

        def dy_chunk(c):
            lo, hi = c * fc, (c + 1) * fc
            return jnp.where(is_x0, dy_ref[:, lo:hi],
                             dy_ref[:, f_half + lo:f_half + hi])

        rdma1 = []
        for c in range(NC):
            c_send[c] = lax.dot_general(
                x_send, dy_chunk(c), (((0,), (0,)), ((), ())),
                preferred_element_type=jnp.float32)
            r = pltpu.make_async_remote_copy(
                src_ref=c_send.at[c], dst_ref=rs_recv.at[c],
                send_sem=sems1_s.at[c], recv_sem=sems1_r.at[c],
                device_id=(my_x, 1 - my_y),
                device_id_type=pl.DeviceIdType.MESH)
            r.start()
            rdma1.append(r)

        rdma2 = []
        for c in range(NC):
            keep = lax.dot_general(
                x_keep, dy_chunk(c), (((0,), (0,)), ((), ())),
                preferred_element_type=jnp.float32)
            rdma1[c].wait()
            r_ref[c] = keep + rs_recv[c]
            r2 = pltpu.make_async_remote_copy(
                src_ref=r_ref.at[c], dst_ref=ag_recv.at[c],
                send_sem=sems2_s.at[c], recv_sem=sems2_r.at[c],
                device_id=(1 - my_x, my_y),
                device_id_type=pl.DeviceIdType.MESH)
            r2.start()
            rdma2.append(r2)

            lo, hi = c * fc, (c + 1) * fc

            @pl.when(is_x0)
            def _(lo=lo, hi=hi, c=c):
                out_ref[:, lo:hi] = r_ref[c]

            @pl.when(~is_x0)
            def _(lo=lo, hi=hi, c=c):
                out_ref[:, f_half + lo:f_half + hi] = r_ref[c]

        for c in range(NC):
            rdma2[c].wait()
            lo, hi = c * fc, (c + 1) * fc

            @pl.when(is_x0)
            def _(lo=lo, hi=hi, c=c):
                out_ref[:, f_half + lo:f_half + hi] = ag_recv[c]

            @pl.when(~is_x0)
            def _(lo=lo, hi=hi, c=c):
                out_ref[:, lo:hi] = ag_recv[c]

    return pl.pallas_call(
        body,
        out_shape=jax.ShapeDtypeStruct((m_half, f), jnp.float32),
        in_specs=[pl.BlockSpec(memory_space=pltpu.VMEM),
                  pl.BlockSpec(memory_space=pltpu.VMEM)],
        out_specs=pl.BlockSpec(memory_space=pltpu.VMEM),
        scratch_shapes=[
            pltpu.VMEM((NC, m_half, fc), jnp.float32),
            pltpu.VMEM((NC, m_half, fc), jnp.float32),
            pltpu.VMEM((NC, m_half, fc), jnp.float32),
            pltpu.VMEM((NC, m_half, fc), jnp.float32),
            pltpu.SemaphoreType.DMA((NC,)),
            pltpu.SemaphoreType.DMA((NC,)),
            pltpu.SemaphoreType.DMA((NC,)),
            pltpu.SemaphoreType.DMA((NC,)),
        ],
        compiler_params=pltpu.CompilerParams(collective_id=0),
    )(x, dy)
